# baseline (speedup 1.0000x reference)
import functools

import jax
import jax.numpy as jnp
from jax import lax
from jax.experimental import pallas as pl
from jax.experimental.pallas import tpu as pltpu
from jax.experimental.pallas import tpu_sc as plsc

_B, _S, _D = 4, 8192, 1024
_NW = 32
_ROWS_W = _S // _NW
_C = 8
_NCH = _ROWS_W // _C
_E = _C * _D
_EF = _B * _E
_NS = 16


def _sc_body(in_hbm, pos_hbm, out_hbm, pos_v, io, sem_pos, sem_in, sem_out):
    cid = lax.axis_index("c")
    sid = lax.axis_index("s")
    wid = sid * 2 + cid
    base_e = wid * _ROWS_W * _D

    def col(c):
        return pl.multiple_of(base_e + c * _E, _E)

    def fcol(c):
        return pl.multiple_of((wid * _NCH + c) * _EF, _EF)

    def issue_in(q, c):
        pltpu.async_copy(in_hbm.at[pl.ds(fcol(c), _EF)], io.at[sid, q], sem_in[q])
        pltpu.async_copy(pos_hbm.at[pl.ds(col(c), _E)], pos_v.at[q], sem_pos[q])

    def wait_in(p):
        pltpu.make_async_copy(in_hbm.at[pl.ds(0, _EF)], io.at[sid, p], sem_in[p]).wait()
        pltpu.make_async_copy(pos_hbm.at[pl.ds(0, _E)], pos_v.at[p], sem_pos[p]).wait()

    def drain_out(q):
        pltpu.make_async_copy(io.at[sid, q], out_hbm.at[pl.ds(0, _EF)], sem_out[q]).wait()

    def segment(p, c, drain):
        q = 1 - p
        c_next = lax.rem(c + 1, _NCH)
        if drain:
            drain_out(q)
        issue_in(q, c_next)
        wait_in(p)
        pltpu.async_copy(io.at[sid, p], out_hbm.at[pl.ds(fcol(c), _EF)], sem_out[p])

    issue_in(0, 0)
    segment(0, 0, drain=False)
    segment(1, 1, drain=True)

    def loop_body(k, _):
        segment(0, 2 * k, drain=True)
        segment(1, 2 * k + 1, drain=True)
        return 0

    lax.fori_loop(1, _NCH // 2, loop_body, 0)

    wait_in(0)
    drain_out(1)


_sc_add = functools.partial(
    pl.kernel,
    mesh=plsc.VectorSubcoreMesh(core_axis_name="c", subcore_axis_name="s"),
    out_type=jax.ShapeDtypeStruct((_B * _S * _D,), jnp.float32),
    scratch_types=[
        pltpu.VMEM((2, _E), jnp.float32),
        pltpu.VMEM_SHARED((_NS, 2, _EF), jnp.float32),
        [pltpu.SemaphoreType.DMA] * 2,
        [pltpu.SemaphoreType.DMA] * 2,
        [pltpu.SemaphoreType.DMA] * 2,
    ],
)(_sc_body)


def kernel(inputs, pos_table):
    B, S, D = inputs.shape
    out = _sc_add(inputs.reshape(B * S * D), pos_table.reshape(S * D))
    return out.reshape(B, S, D)


# speedup vs baseline: 4.0308x; 4.0308x over previous
import jax
import jax.numpy as jnp
from jax.experimental import pallas as pl


_BS = 512


def _add_body(in_ref, pos_ref, out_ref):
    out_ref[...] = in_ref[...] + pos_ref[...][None, :, :]


def kernel(inputs, pos_table):
    B, S, D = inputs.shape
    return pl.pallas_call(
        _add_body,
        grid=(S // _BS,),
        in_specs=[
            pl.BlockSpec((B, _BS, D), lambda i: (0, i, 0)),
            pl.BlockSpec((_BS, D), lambda i: (i, 0)),
        ],
        out_specs=pl.BlockSpec((B, _BS, D), lambda i: (0, i, 0)),
        out_shape=jax.ShapeDtypeStruct((B, S, D), inputs.dtype),
    )(inputs, pos_table)
